# baseline (speedup 1.0000x reference)
import functools

import jax
import jax.numpy as jnp
from jax import lax
from jax.experimental import pallas as pl
from jax.experimental.pallas import tpu as pltpu
from jax.experimental.pallas import tpu_sc as plsc

N = 50000
T = 8
F = 128
H = 64
E = 800000

NS = 16
L = 16
CH = 128
QD = 4
NCH = 392
NG = NCH
ET_PAD = NCH * CH
E_PAD = ET_PAD * NS
ZCH = 25
N_ACC = NS * ZCH * CH
DEG_HG = NCH // QD // 2

B = 2000
NBLK = N // B

_MESH = dict(core_axis_name="c", subcore_axis_name="s")


@functools.partial(
    pl.kernel,
    mesh=plsc.VectorSubcoreMesh(**_MESH),
    out_type=jax.ShapeDtypeStruct((2, N_ACC, L), jnp.float32),
    scratch_types=[
        pltpu.VMEM((QD, CH), jnp.int32),
        pltpu.VMEM((CH, L), jnp.float32),
        pltpu.VMEM((CH, L), jnp.float32),
        pltpu.VMEM_SHARED((N_ACC, L), jnp.float32),
    ],
    compiler_params=pltpu.CompilerParams(use_tc_tiling_on_sc=False),
)
def _deg_kernel(dst_hbm, out_hbm, dst_v, ones_v, zero_v, acc_sh):
    c = lax.axis_index("c")
    s = lax.axis_index("s")

    lane = lax.iota(jnp.int32, L)
    one_hot = jnp.where(lane == 0, 1.0, 0.0).astype(jnp.float32)
    zeros = jnp.zeros((L,), jnp.float32)

    def _fill(r, _):
        ones_v[r, :] = one_hot
        zero_v[r, :] = zeros
        return 0

    lax.fori_loop(0, CH, _fill, 0)

    def _zcopy(z, _):
        pltpu.sync_copy(zero_v, acc_sh.at[pl.ds((s * ZCH + z) * CH, CH)])
        return 0

    lax.fori_loop(0, ZCH, _zcopy, 0)
    plsc.subcore_barrier()

    def _group(g, _):
        pltpu.sync_copy(dst_hbm.at[s, pl.ds(g * QD, QD)], dst_v)
        for q in range(QD):
            pltpu.sync_copy(ones_v, acc_sh.at[dst_v.at[q]], add=True)
        return 0

    lax.fori_loop(c * DEG_HG, (c + 1) * DEG_HG, _group, 0)
    plsc.subcore_barrier()

    def _wb(z, _):
        r0 = (s * ZCH + z) * CH
        pltpu.sync_copy(acc_sh.at[pl.ds(r0, CH)], zero_v)
        pltpu.sync_copy(zero_v, out_hbm.at[c, pl.ds(r0, CH)])
        return 0

    lax.fori_loop(0, ZCH, _wb, 0)


def _make_seg_sum(dh):
    GC = CH
    NQ = NG // 4

    @functools.partial(
        pl.kernel,
        mesh=plsc.VectorSubcoreMesh(**_MESH),
        out_type=jax.ShapeDtypeStruct((2, N_ACC, dh), jnp.float32),
        scratch_types=[
            [pltpu.VMEM((GC,), jnp.int32) for _ in range(4)],
            [pltpu.VMEM((1, CH), jnp.int32) for _ in range(4)],
            [pltpu.VMEM((GC, dh), jnp.float32) for _ in range(2)],
            pltpu.VMEM_SHARED((N_ACC, dh), jnp.float32),
            pltpu.SemaphoreType.DMA,
            pltpu.SemaphoreType.DMA,
            pltpu.SemaphoreType.DMA,
        ],
        compiler_params=pltpu.CompilerParams(use_tc_tiling_on_sc=False),
    )
    def seg_sum(yw_hbm, src_hbm, dst_hbm, out_hbm, src_v, dst_v, rows_v, acc_sh,
                sem_g, sem_s, sem_i):
        c = lax.axis_index("c")
        s = lax.axis_index("s")

        zeros = jnp.zeros((L,), jnp.float32)
        kpr = dh // L
        r0v = rows_v[0]

        def _zrow(i, _):
            r0v[i // kpr, pl.ds((i % kpr) * L, L)] = zeros
            return 0

        lax.fori_loop(0, CH * kpr, _zrow, 0)

        def _zcopy(z, _):
            pltpu.sync_copy(r0v.at[pl.ds(0, CH)],
                            acc_sh.at[pl.ds((s * ZCH + z) * CH, CH)])
            return 0

        lax.fori_loop(0, ZCH, _zcopy, 0)
        plsc.subcore_barrier()

        def _stage(slot, g):
            e0 = s * ET_PAD + g * GC
            pltpu.async_copy(src_hbm.at[c, pl.ds(e0, GC)], src_v[slot], sem_i)
            pltpu.async_copy(dst_hbm.at[s, pl.ds(g, 1)], dst_v[slot], sem_i)

        def _stage_wait(slot):
            pltpu.make_async_copy(src_hbm.at[c, pl.ds(0, GC)], src_v[slot],
                                  sem_i).wait()
            pltpu.make_async_copy(dst_hbm.at[s, pl.ds(0, 1)], dst_v[slot],
                                  sem_i).wait()

        def _gather(slot, rb):
            pltpu.async_copy(yw_hbm.at[src_v[slot]], rows_v[rb], sem_g)

        def _gather_wait(slot, rb):
            pltpu.make_async_copy(yw_hbm.at[src_v[slot]], rows_v[rb], sem_g).wait()

        def _scatter(slot, rb):
            return [
                pltpu.async_copy(rows_v[rb],
                                 acc_sh.at[dst_v[slot].at[0]], sem_s, add=True)
            ]

        for slot in range(4):
            _stage(slot, slot)
        _stage_wait(0)
        _gather(0, 0)

        def _quad(p, _):
            q0 = 4 * p
            for k in range(4):
                slot, nslot, rb = k, (k + 1) % 4, k % 2
                _gather_wait(slot, rb)
                if k < 3:
                    _stage_wait(nslot)
                    _gather(nslot, 1 - rb)
                sc = _scatter(slot, rb)
                for h in sc:
                    h.wait()

                @pl.when(q0 + k + 4 < NG)
                def _(slot=slot, k=k, q0=q0):
                    _stage(slot, q0 + k + 4)

                if k == 3:
                    @pl.when(q0 + 4 < NG)
                    def _(q0=q0):
                        _stage_wait(0)
                        _gather(0, 0)
            return 0

        lax.fori_loop(0, NQ, _quad, 0)
        plsc.subcore_barrier()

        def _wb(z, _):
            r0 = (s * ZCH + z) * CH
            pltpu.sync_copy(acc_sh.at[pl.ds(r0, CH)], r0v.at[pl.ds(0, CH)])
            pltpu.sync_copy(r0v.at[pl.ds(0, CH)], out_hbm.at[c, pl.ds(r0, CH)])
            return 0

        lax.fori_loop(0, ZCH, _wb, 0)

    return seg_sum

    return seg_sum


_seg_sum_32 = _make_seg_sum(32)
_seg_sum_16 = _make_seg_sum(16)


def _lstm_body(x_ref, wih_ref, whh_ref, bih_ref, bhh_ref, h_ref):
    bias = bih_ref[:] + bhh_ref[:]
    h = jnp.zeros((B, H), jnp.float32)
    c = jnp.zeros((B, H), jnp.float32)
    for t in range(T):
        g = jnp.dot(x_ref[:, t, :], wih_ref[:], preferred_element_type=jnp.float32)
        g = g + jnp.dot(h, whh_ref[:], preferred_element_type=jnp.float32)
        g = g + bias
        i = jax.nn.sigmoid(g[:, 0:H])
        f = jax.nn.sigmoid(g[:, H:2 * H])
        gg = jnp.tanh(g[:, 2 * H:3 * H])
        o = jax.nn.sigmoid(g[:, 3 * H:4 * H])
        c = f * c + i * gg
        h = o * jnp.tanh(c)
    h_ref[:] = h


def _lstm_call(x, wih_t, whh_t, bih, bhh):
    return pl.pallas_call(
        _lstm_body,
        grid=(NBLK,),
        in_specs=[
            pl.BlockSpec((B, T, F), lambda i: (i, 0, 0)),
            pl.BlockSpec((F, 4 * H), lambda i: (0, 0)),
            pl.BlockSpec((H, 4 * H), lambda i: (0, 0)),
            pl.BlockSpec((1, 4 * H), lambda i: (0, 0)),
            pl.BlockSpec((1, 4 * H), lambda i: (0, 0)),
        ],
        out_specs=pl.BlockSpec((B, H), lambda i: (i, 0)),
        out_shape=jax.ShapeDtypeStruct((N, H), jnp.float32),
    )(x, wih_t, whh_t, bih, bhh)


def _prep1_body(h_ref, p0_ref, p1_ref, w_ref, yw_ref, dinv_ref):
    deg = p0_ref[:, 0:1] + p1_ref[:, 0:1] + 1.0
    dinv = lax.rsqrt(deg)
    yw = jnp.dot(dinv * h_ref[:], w_ref[:], preferred_element_type=jnp.float32)
    yw_ref[0] = yw[:, 0:H // 2]
    yw_ref[1] = yw[:, H // 2:H]
    dinv_ref[:] = jnp.broadcast_to(dinv, (B, L))


def _prep1_call(h, p0, p1, w1_t):
    return pl.pallas_call(
        _prep1_body,
        grid=(NBLK,),
        in_specs=[
            pl.BlockSpec((B, H), lambda i: (i, 0)),
            pl.BlockSpec((B, L), lambda i: (i, 0)),
            pl.BlockSpec((B, L), lambda i: (i, 0)),
            pl.BlockSpec((H, H), lambda i: (0, 0)),
        ],
        out_specs=[
            pl.BlockSpec((2, B, H // 2), lambda i: (0, i, 0)),
            pl.BlockSpec((B, L), lambda i: (i, 0)),
        ],
        out_shape=[
            jax.ShapeDtypeStruct((2, N, H // 2), jnp.float32),
            jax.ShapeDtypeStruct((N, L), jnp.float32),
        ],
    )(h, p0, p1, w1_t)


def _comb1_body(slo_ref, shi_ref, ylo_ref, yhi_ref, dinv_ref, b1_ref, w_ref, out_ref):
    dinv = dinv_ref[:, 0:1]
    zlo = jnp.maximum(dinv * (slo_ref[:] + ylo_ref[:]) + b1_ref[:, 0:H // 2], 0.0)
    zhi = jnp.maximum(dinv * (shi_ref[:] + yhi_ref[:]) + b1_ref[:, H // 2:H], 0.0)
    z = jnp.concatenate([zlo, zhi], axis=1)
    yw2 = jnp.dot(dinv * z, w_ref[:], preferred_element_type=jnp.float32)
    out_ref[0] = yw2[:, 0:H // 4]
    out_ref[1] = yw2[:, H // 4:H // 2]


def _comb1_call(slo, shi, ylo, yhi, dinv16, b1, w2_t):
    return pl.pallas_call(
        _comb1_body,
        grid=(NBLK,),
        in_specs=[
            pl.BlockSpec((B, H // 2), lambda i: (i, 0)),
            pl.BlockSpec((B, H // 2), lambda i: (i, 0)),
            pl.BlockSpec((B, H // 2), lambda i: (i, 0)),
            pl.BlockSpec((B, H // 2), lambda i: (i, 0)),
            pl.BlockSpec((B, L), lambda i: (i, 0)),
            pl.BlockSpec((1, H), lambda i: (0, 0)),
            pl.BlockSpec((H, H // 2), lambda i: (0, 0)),
        ],
        out_specs=pl.BlockSpec((2, B, H // 4), lambda i: (0, i, 0)),
        out_shape=jax.ShapeDtypeStruct((2, N, H // 4), jnp.float32),
    )(slo, shi, ylo, yhi, dinv16, b1, w2_t)


def _comb2_body(slo_ref, shi_ref, ylo_ref, yhi_ref, dinv_ref, b2_ref, wc_ref,
                bc_ref, out_ref):
    dinv = dinv_ref[:, 0:1]
    zlo = jnp.maximum(dinv * (slo_ref[:] + ylo_ref[:]) + b2_ref[:, 0:H // 4], 0.0)
    zhi = jnp.maximum(dinv * (shi_ref[:] + yhi_ref[:]) + b2_ref[:, H // 4:H // 2], 0.0)
    z = jnp.concatenate([zlo, zhi], axis=1)
    logits = jnp.dot(z, wc_ref[:], preferred_element_type=jnp.float32) + bc_ref[:]
    m = jnp.max(logits, axis=1, keepdims=True)
    e = jnp.exp(logits - m)
    out_ref[:] = e / jnp.sum(e, axis=1, keepdims=True)


def _comb2_call(slo, shi, ylo, yhi, dinv16, b2, wc_t, bc):
    return pl.pallas_call(
        _comb2_body,
        grid=(NBLK,),
        in_specs=[
            pl.BlockSpec((B, H // 4), lambda i: (i, 0)),
            pl.BlockSpec((B, H // 4), lambda i: (i, 0)),
            pl.BlockSpec((B, H // 4), lambda i: (i, 0)),
            pl.BlockSpec((B, H // 4), lambda i: (i, 0)),
            pl.BlockSpec((B, L), lambda i: (i, 0)),
            pl.BlockSpec((1, H // 2), lambda i: (0, 0)),
            pl.BlockSpec((H // 2, 2), lambda i: (0, 0)),
            pl.BlockSpec((1, 2), lambda i: (0, 0)),
        ],
        out_specs=pl.BlockSpec((B, 2), lambda i: (i, 0)),
        out_shape=jax.ShapeDtypeStruct((N, 2), jnp.float32),
    )(slo, shi, ylo, yhi, dinv16, b2, wc_t, bc)


def kernel(x, edge_index, W_ih, W_hh, b_ih, b_hh, W1, b1, W2, b2, Wc, bc):
    src = edge_index[0].astype(jnp.int32)
    dst = edge_index[1].astype(jnp.int32)
    pad = E_PAD - E
    pad_pos = jnp.arange(pad, dtype=jnp.int32)
    srcp = jnp.concatenate([src, pad_pos % N])
    dstp = jnp.concatenate([dst, N + pad_pos % (N_ACC - N)])
    src2 = jnp.stack([srcp, srcp + N])
    dst3 = dstp.reshape(NS, NCH, CH)

    deg_parts = _deg_kernel(dst3)
    p0 = deg_parts[0, :N]
    p1 = deg_parts[1, :N]

    h = _lstm_call(x, W_ih.T, W_hh.T, b_ih.reshape(1, -1), b_hh.reshape(1, -1))
    yw1_s, dinv16 = _prep1_call(h, p0, p1, W1.T)
    seg1 = _seg_sum_32(yw1_s.reshape(2 * N, H // 2), src2, dst3)
    yw2_s = _comb1_call(seg1[0, :N], seg1[1, :N], yw1_s[0], yw1_s[1],
                        dinv16, b1.reshape(1, -1), W2.T)
    seg2 = _seg_sum_16(yw2_s.reshape(2 * N, H // 4), src2, dst3)
    return _comb2_call(seg2[0, :N], seg2[1, :N], yw2_s[0], yw2_s[1],
                       dinv16, b2.reshape(1, -1), Wc.T, bc.reshape(1, -1))


# speedup vs baseline: 1.0955x; 1.0955x over previous
import functools

import jax
import jax.numpy as jnp
from jax import lax
from jax.experimental import pallas as pl
from jax.experimental.pallas import tpu as pltpu
from jax.experimental.pallas import tpu_sc as plsc

N = 50000
T = 8
F = 128
H = 64
E = 800000

NS = 16
L = 16
CH = 128
QD = 4
NCH = 392
NG = NCH
ET_PAD = NCH * CH
E_PAD = ET_PAD * NS
ZCH = 25
N_ACC = NS * ZCH * CH
DEG_HG = NCH // QD // 2

B = 2000
NBLK = N // B

_MESH = dict(core_axis_name="c", subcore_axis_name="s")


@functools.partial(
    pl.kernel,
    mesh=plsc.VectorSubcoreMesh(**_MESH),
    out_type=jax.ShapeDtypeStruct((2, N_ACC, L), jnp.float32),
    scratch_types=[
        pltpu.VMEM((QD, CH), jnp.int32),
        pltpu.VMEM((CH, L), jnp.float32),
        pltpu.VMEM((CH, L), jnp.float32),
        pltpu.VMEM_SHARED((N_ACC, L), jnp.float32),
    ],
    compiler_params=pltpu.CompilerParams(use_tc_tiling_on_sc=False),
)
def _deg_kernel(dst_hbm, out_hbm, dst_v, ones_v, zero_v, acc_sh):
    c = lax.axis_index("c")
    s = lax.axis_index("s")

    lane = lax.iota(jnp.int32, L)
    one_hot = jnp.where(lane == 0, 1.0, 0.0).astype(jnp.float32)
    zeros = jnp.zeros((L,), jnp.float32)

    def _fill(r, _):
        ones_v[r, :] = one_hot
        zero_v[r, :] = zeros
        return 0

    lax.fori_loop(0, CH, _fill, 0)

    def _zcopy(z, _):
        pltpu.sync_copy(zero_v, acc_sh.at[pl.ds((s * ZCH + z) * CH, CH)])
        return 0

    lax.fori_loop(0, ZCH, _zcopy, 0)
    plsc.subcore_barrier()

    def _group(g, _):
        pltpu.sync_copy(dst_hbm.at[s, pl.ds(g * QD, QD)], dst_v)
        for q in range(QD):
            pltpu.sync_copy(ones_v, acc_sh.at[dst_v.at[q]], add=True)
        return 0

    lax.fori_loop(c * DEG_HG, (c + 1) * DEG_HG, _group, 0)
    plsc.subcore_barrier()

    def _wb(z, _):
        r0 = (s * ZCH + z) * CH
        pltpu.sync_copy(acc_sh.at[pl.ds(r0, CH)], zero_v)
        pltpu.sync_copy(zero_v, out_hbm.at[c, pl.ds(r0, CH)])
        return 0

    lax.fori_loop(0, ZCH, _wb, 0)


def _make_seg_sum(dh):
    GC = CH
    NQ = NG // 4

    @functools.partial(
        pl.kernel,
        mesh=plsc.VectorSubcoreMesh(**_MESH),
        out_type=jax.ShapeDtypeStruct((2, N_ACC, dh), jnp.float32),
        scratch_types=[
            [pltpu.VMEM((GC,), jnp.int32) for _ in range(4)],
            [pltpu.VMEM((1, CH), jnp.int32) for _ in range(4)],
            [pltpu.VMEM((GC, dh), jnp.float32) for _ in range(2)],
            pltpu.VMEM_SHARED((N_ACC, dh), jnp.float32),
            pltpu.SemaphoreType.DMA,
            pltpu.SemaphoreType.DMA,
            pltpu.SemaphoreType.DMA,
        ],
        compiler_params=pltpu.CompilerParams(use_tc_tiling_on_sc=False),
    )
    def seg_sum(yw_hbm, src_hbm, dst_hbm, out_hbm, src_v, dst_v, rows_v, acc_sh,
                sem_g, sem_s, sem_i):
        c = lax.axis_index("c")
        s = lax.axis_index("s")

        zeros = jnp.zeros((L,), jnp.float32)
        kpr = dh // L
        r0v = rows_v[0]

        def _zrow(i, _):
            r0v[i // kpr, pl.ds((i % kpr) * L, L)] = zeros
            return 0

        lax.fori_loop(0, CH * kpr, _zrow, 0)

        def _zcopy(z, _):
            pltpu.sync_copy(r0v.at[pl.ds(0, CH)],
                            acc_sh.at[pl.ds((s * ZCH + z) * CH, CH)])
            return 0

        lax.fori_loop(0, ZCH, _zcopy, 0)
        plsc.subcore_barrier()

        def _stage(slot, g):
            e0 = s * ET_PAD + g * GC
            pltpu.async_copy(src_hbm.at[c, pl.ds(e0, GC)], src_v[slot], sem_i)
            pltpu.async_copy(dst_hbm.at[s, pl.ds(g, 1)], dst_v[slot], sem_i)

        def _stage_wait(slot):
            pltpu.make_async_copy(src_hbm.at[c, pl.ds(0, GC)], src_v[slot],
                                  sem_i).wait()
            pltpu.make_async_copy(dst_hbm.at[s, pl.ds(0, 1)], dst_v[slot],
                                  sem_i).wait()

        def _gather(slot, rb):
            pltpu.async_copy(yw_hbm.at[src_v[slot]], rows_v[rb], sem_g)

        def _gather_wait(slot, rb):
            pltpu.make_async_copy(yw_hbm.at[src_v[slot]], rows_v[rb], sem_g).wait()

        def _scatter(slot, rb):
            return [
                pltpu.async_copy(rows_v[rb],
                                 acc_sh.at[dst_v[slot].at[0]], sem_s, add=True)
            ]

        for slot in range(4):
            _stage(slot, slot)
        _stage_wait(0)
        _gather(0, 0)

        def _quad(p, _):
            q0 = 4 * p
            for k in range(4):
                slot, nslot, rb = k, (k + 1) % 4, k % 2
                _gather_wait(slot, rb)
                if k < 3:
                    _stage_wait(nslot)
                    _gather(nslot, 1 - rb)
                sc = _scatter(slot, rb)
                for h in sc:
                    h.wait()

                @pl.when(q0 + k + 4 < NG)
                def _(slot=slot, k=k, q0=q0):
                    _stage(slot, q0 + k + 4)

                if k == 3:
                    @pl.when(q0 + 4 < NG)
                    def _(q0=q0):
                        _stage_wait(0)
                        _gather(0, 0)
            return 0

        lax.fori_loop(0, NQ, _quad, 0)
        plsc.subcore_barrier()

        def _wb(z, _):
            r0 = (s * ZCH + z) * CH
            pltpu.sync_copy(acc_sh.at[pl.ds(r0, CH)], r0v.at[pl.ds(0, CH)])
            pltpu.sync_copy(r0v.at[pl.ds(0, CH)], out_hbm.at[c, pl.ds(r0, CH)])
            return 0

        lax.fori_loop(0, ZCH, _wb, 0)

    return seg_sum

    return seg_sum


_seg_sum_32 = _make_seg_sum(32)
_seg_sum_16 = _make_seg_sum(16)


def _lstm_body(x_ref, wih_ref, whh_ref, bih_ref, bhh_ref, p0_ref, p1_ref,
               w1_ref, yw_ref, dinv_ref):
    bias = bih_ref[:] + bhh_ref[:]
    h = jnp.zeros((B, H), jnp.float32)
    c = jnp.zeros((B, H), jnp.float32)
    for t in range(T):
        g = jnp.dot(x_ref[:, t, :], wih_ref[:], preferred_element_type=jnp.float32)
        g = g + jnp.dot(h, whh_ref[:], preferred_element_type=jnp.float32)
        g = g + bias
        i = jax.nn.sigmoid(g[:, 0:H])
        f = jax.nn.sigmoid(g[:, H:2 * H])
        gg = jnp.tanh(g[:, 2 * H:3 * H])
        o = jax.nn.sigmoid(g[:, 3 * H:4 * H])
        c = f * c + i * gg
        h = o * jnp.tanh(c)
    deg = p0_ref[0, :, 0:1] + p1_ref[0, :, 0:1] + 1.0
    dinv = lax.rsqrt(deg)
    yw = jnp.dot(dinv * h, w1_ref[:], preferred_element_type=jnp.float32)
    yw_ref[0] = yw[:, 0:H // 2]
    yw_ref[1] = yw[:, H // 2:H]
    dinv_ref[:] = jnp.broadcast_to(dinv, (B, L))


def _lstm_call(x, wih_t, whh_t, bih, bhh, deg_parts, w1_t):
    return pl.pallas_call(
        _lstm_body,
        grid=(NBLK,),
        in_specs=[
            pl.BlockSpec((B, T, F), lambda i: (i, 0, 0)),
            pl.BlockSpec((F, 4 * H), lambda i: (0, 0)),
            pl.BlockSpec((H, 4 * H), lambda i: (0, 0)),
            pl.BlockSpec((1, 4 * H), lambda i: (0, 0)),
            pl.BlockSpec((1, 4 * H), lambda i: (0, 0)),
            pl.BlockSpec((1, B, L), lambda i: (0, i, 0)),
            pl.BlockSpec((1, B, L), lambda i: (1, i, 0)),
            pl.BlockSpec((H, H), lambda i: (0, 0)),
        ],
        out_specs=[
            pl.BlockSpec((2, B, H // 2), lambda i: (0, i, 0)),
            pl.BlockSpec((B, L), lambda i: (i, 0)),
        ],
        out_shape=[
            jax.ShapeDtypeStruct((2, N, H // 2), jnp.float32),
            jax.ShapeDtypeStruct((N, L), jnp.float32),
        ],
    )(x, wih_t, whh_t, bih, bhh, deg_parts, deg_parts, w1_t)


def _comb1_body(slo_ref, shi_ref, ylo_ref, yhi_ref, dinv_ref, b1_ref, w_ref, out_ref):
    dinv = dinv_ref[:, 0:1]
    zlo = jnp.maximum(dinv * (slo_ref[0] + ylo_ref[0]) + b1_ref[:, 0:H // 2], 0.0)
    zhi = jnp.maximum(dinv * (shi_ref[0] + yhi_ref[0]) + b1_ref[:, H // 2:H], 0.0)
    z = jnp.concatenate([zlo, zhi], axis=1)
    yw2 = jnp.dot(dinv * z, w_ref[:], preferred_element_type=jnp.float32)
    out_ref[0] = yw2[:, 0:H // 4]
    out_ref[1] = yw2[:, H // 4:H // 2]


def _comb1_call(seg1, yw1_s, dinv16, b1, w2_t):
    return pl.pallas_call(
        _comb1_body,
        grid=(NBLK,),
        in_specs=[
            pl.BlockSpec((1, B, H // 2), lambda i: (0, i, 0)),
            pl.BlockSpec((1, B, H // 2), lambda i: (1, i, 0)),
            pl.BlockSpec((1, B, H // 2), lambda i: (0, i, 0)),
            pl.BlockSpec((1, B, H // 2), lambda i: (1, i, 0)),
            pl.BlockSpec((B, L), lambda i: (i, 0)),
            pl.BlockSpec((1, H), lambda i: (0, 0)),
            pl.BlockSpec((H, H // 2), lambda i: (0, 0)),
        ],
        out_specs=pl.BlockSpec((2, B, H // 4), lambda i: (0, i, 0)),
        out_shape=jax.ShapeDtypeStruct((2, N, H // 4), jnp.float32),
    )(seg1, seg1, yw1_s, yw1_s, dinv16, b1, w2_t)


def _comb2_body(slo_ref, shi_ref, ylo_ref, yhi_ref, dinv_ref, b2_ref, wc_ref,
                bc_ref, out_ref):
    dinv = dinv_ref[:, 0:1]
    zlo = jnp.maximum(dinv * (slo_ref[0] + ylo_ref[0]) + b2_ref[:, 0:H // 4], 0.0)
    zhi = jnp.maximum(dinv * (shi_ref[0] + yhi_ref[0]) + b2_ref[:, H // 4:H // 2], 0.0)
    z = jnp.concatenate([zlo, zhi], axis=1)
    logits = jnp.dot(z, wc_ref[:], preferred_element_type=jnp.float32) + bc_ref[:]
    m = jnp.max(logits, axis=1, keepdims=True)
    e = jnp.exp(logits - m)
    out_ref[:] = e / jnp.sum(e, axis=1, keepdims=True)


def _comb2_call(seg2, yw2_s, dinv16, b2, wc_t, bc):
    return pl.pallas_call(
        _comb2_body,
        grid=(NBLK,),
        in_specs=[
            pl.BlockSpec((1, B, H // 4), lambda i: (0, i, 0)),
            pl.BlockSpec((1, B, H // 4), lambda i: (1, i, 0)),
            pl.BlockSpec((1, B, H // 4), lambda i: (0, i, 0)),
            pl.BlockSpec((1, B, H // 4), lambda i: (1, i, 0)),
            pl.BlockSpec((B, L), lambda i: (i, 0)),
            pl.BlockSpec((1, H // 2), lambda i: (0, 0)),
            pl.BlockSpec((H // 2, 2), lambda i: (0, 0)),
            pl.BlockSpec((1, 2), lambda i: (0, 0)),
        ],
        out_specs=pl.BlockSpec((B, 2), lambda i: (i, 0)),
        out_shape=jax.ShapeDtypeStruct((N, 2), jnp.float32),
    )(seg2, seg2, yw2_s, yw2_s, dinv16, b2, wc_t, bc)


def kernel(x, edge_index, W_ih, W_hh, b_ih, b_hh, W1, b1, W2, b2, Wc, bc):
    src = edge_index[0].astype(jnp.int32)
    dst = edge_index[1].astype(jnp.int32)
    pad = E_PAD - E
    pad_pos = jnp.arange(pad, dtype=jnp.int32)
    srcp = jnp.concatenate([src, pad_pos % N])
    dstp = jnp.concatenate([dst, N + pad_pos % (N_ACC - N)])
    src2 = jnp.stack([srcp, srcp + N])
    dst3 = dstp.reshape(NS, NCH, CH)

    deg_parts = _deg_kernel(dst3)

    yw1_s, dinv16 = _lstm_call(x, W_ih.T, W_hh.T, b_ih.reshape(1, -1),
                               b_hh.reshape(1, -1), deg_parts, W1.T)
    seg1 = _seg_sum_32(yw1_s.reshape(2 * N, H // 2), src2, dst3)
    yw2_s = _comb1_call(seg1, yw1_s, dinv16, b1.reshape(1, -1), W2.T)
    seg2 = _seg_sum_16(yw2_s.reshape(2 * N, H // 4), src2, dst3)
    return _comb2_call(seg2, yw2_s, dinv16, b2.reshape(1, -1), Wc.T,
                       bc.reshape(1, -1))


# speedup vs baseline: 1.1327x; 1.0340x over previous
import functools

import jax
import jax.numpy as jnp
from jax import lax
from jax.experimental import pallas as pl
from jax.experimental.pallas import tpu as pltpu
from jax.experimental.pallas import tpu_sc as plsc

N = 50000
T = 8
F = 128
H = 64
E = 800000

NS = 16
L = 16
CH = 128
QD = 4
NCH = 392
NG = NCH
ET_PAD = NCH * CH
E_PAD = ET_PAD * NS
ZCH = 25
N_ACC = NS * ZCH * CH
DEG_HG = NCH // QD // 2

B = 2000
NBLK = N // B

_MESH = dict(core_axis_name="c", subcore_axis_name="s")


@functools.partial(
    pl.kernel,
    mesh=plsc.VectorSubcoreMesh(**_MESH),
    out_type=jax.ShapeDtypeStruct((2, N_ACC, L), jnp.float32),
    scratch_types=[
        pltpu.VMEM((QD, CH), jnp.int32),
        pltpu.VMEM((CH, L), jnp.float32),
        pltpu.VMEM((CH, L), jnp.float32),
        pltpu.VMEM_SHARED((N_ACC, L), jnp.float32),
    ],
    compiler_params=pltpu.CompilerParams(use_tc_tiling_on_sc=False),
)
def _deg_kernel(dst_hbm, out_hbm, dst_v, ones_v, zero_v, acc_sh):
    c = lax.axis_index("c")
    s = lax.axis_index("s")

    lane = lax.iota(jnp.int32, L)
    one_hot = jnp.where(lane == 0, 1.0, 0.0).astype(jnp.float32)
    zeros = jnp.zeros((L,), jnp.float32)

    def _fill(r, _):
        ones_v[r, :] = one_hot
        zero_v[r, :] = zeros
        return 0

    lax.fori_loop(0, CH, _fill, 0)

    def _zcopy(z, _):
        pltpu.sync_copy(zero_v, acc_sh.at[pl.ds((s * ZCH + z) * CH, CH)])
        return 0

    lax.fori_loop(0, ZCH, _zcopy, 0)
    plsc.subcore_barrier()

    def _group(g, _):
        pltpu.sync_copy(dst_hbm.at[s, pl.ds(g * QD, QD)], dst_v)
        for q in range(QD):
            pltpu.sync_copy(ones_v, acc_sh.at[dst_v.at[q]], add=True)
        return 0

    lax.fori_loop(c * DEG_HG, (c + 1) * DEG_HG, _group, 0)
    plsc.subcore_barrier()

    def _wb(z, _):
        r0 = (s * ZCH + z) * CH
        pltpu.sync_copy(acc_sh.at[pl.ds(r0, CH)], zero_v)
        pltpu.sync_copy(zero_v, out_hbm.at[c, pl.ds(r0, CH)])
        return 0

    lax.fori_loop(0, ZCH, _wb, 0)


def _make_seg_sum(dh):
    NQ = NG // 4

    @functools.partial(
        pl.kernel,
        mesh=plsc.VectorSubcoreMesh(**_MESH),
        out_type=jax.ShapeDtypeStruct((2, N_ACC, dh), jnp.float32),
        scratch_types=[
            [pltpu.VMEM((1, 2, CH), jnp.int32) for _ in range(4)],
            [pltpu.VMEM((CH, dh), jnp.float32) for _ in range(2)],
            pltpu.VMEM_SHARED((N_ACC, dh), jnp.float32),
            pltpu.SemaphoreType.DMA,
            pltpu.SemaphoreType.DMA,
            pltpu.SemaphoreType.DMA,
        ],
        compiler_params=pltpu.CompilerParams(use_tc_tiling_on_sc=False),
    )
    def seg_sum(yw_hbm, eidx_hbm, out_hbm, idx_v, rows_v, acc_sh,
                sem_g, sem_s, sem_i):
        c = lax.axis_index("c")
        s = lax.axis_index("s")

        zeros = jnp.zeros((L,), jnp.float32)
        kpr = dh // L
        r0v = rows_v[0]

        def _zrow(i, _):
            r0v[i // kpr, pl.ds((i % kpr) * L, L)] = zeros
            return 0

        lax.fori_loop(0, CH * kpr, _zrow, 0)

        def _zcopy(z, _):
            pltpu.sync_copy(r0v, acc_sh.at[pl.ds((s * ZCH + z) * CH, CH)])
            return 0

        lax.fori_loop(0, ZCH, _zcopy, 0)

        pad0 = jnp.int32(N)
        lane = lax.iota(jnp.int32, L)

        def _prime(j, _):
            idx_v[3][0, 1, pl.ds(j * L, L)] = pad0 + j * L + lane
            return 0

        lax.fori_loop(0, CH // L, _prime, 0)
        plsc.subcore_barrier()

        def _stage(slot, g):
            pltpu.async_copy(eidx_hbm.at[c, s, pl.ds(g, 1)], idx_v[slot], sem_i)

        def _stage_wait(slot):
            pltpu.make_async_copy(eidx_hbm.at[c, s, pl.ds(0, 1)], idx_v[slot],
                                  sem_i).wait()

        def _gather(slot, rb):
            pltpu.async_copy(yw_hbm.at[idx_v[slot].at[0, 0]], rows_v[rb], sem_g)

        def _gather_wait(slot, rb):
            pltpu.make_async_copy(yw_hbm.at[idx_v[slot].at[0, 0]], rows_v[rb],
                                  sem_g).wait()

        def _scatter(slot, rb):
            pltpu.async_copy(rows_v[rb], acc_sh.at[idx_v[slot].at[0, 1]],
                             sem_s, add=True)

        def _scatter_drain(slot, rb):
            pltpu.make_async_copy(rows_v[rb], acc_sh.at[idx_v[slot].at[0, 1]],
                                  sem_s).wait()

        for slot in range(3):
            _stage(slot, slot)
        _scatter(3, 1)
        _stage_wait(0)
        _gather(0, 0)

        def _quad(p, _):
            q0 = 4 * p
            for j in range(4):
                k = q0 + j
                slot, pslot, nslot, rb = j, (j - 1) % 4, (j + 1) % 4, j % 2
                _gather_wait(slot, rb)
                _scatter_drain(pslot, 1 - rb)

                @pl.when(k + 1 < NG)
                def _(nslot=nslot, rb=rb):
                    _stage_wait(nslot)
                    _gather(nslot, 1 - rb)

                _scatter(slot, rb)

                @pl.when(k + 3 < NG)
                def _(pslot=pslot, k=k):
                    _stage(pslot, k + 3)
            return 0

        lax.fori_loop(0, NQ, _quad, 0)
        _scatter_drain(3, 1)
        plsc.subcore_barrier()

        def _wb(z, _):
            r0 = (s * ZCH + z) * CH
            pltpu.sync_copy(acc_sh.at[pl.ds(r0, CH)], r0v)
            pltpu.sync_copy(r0v, out_hbm.at[c, pl.ds(r0, CH)])
            return 0

        lax.fori_loop(0, ZCH, _wb, 0)

    return seg_sum


_seg_sum_32 = _make_seg_sum(32)
_seg_sum_16 = _make_seg_sum(16)


def _lstm_body(x_ref, wih_ref, whh_ref, bih_ref, bhh_ref, p0_ref, p1_ref,
               w1_ref, yw_ref, dinv_ref):
    bias = bih_ref[:] + bhh_ref[:]
    h = jnp.zeros((B, H), jnp.float32)
    c = jnp.zeros((B, H), jnp.float32)
    for t in range(T):
        g = jnp.dot(x_ref[:, t, :], wih_ref[:], preferred_element_type=jnp.float32)
        g = g + jnp.dot(h, whh_ref[:], preferred_element_type=jnp.float32)
        g = g + bias
        i = jax.nn.sigmoid(g[:, 0:H])
        f = jax.nn.sigmoid(g[:, H:2 * H])
        gg = jnp.tanh(g[:, 2 * H:3 * H])
        o = jax.nn.sigmoid(g[:, 3 * H:4 * H])
        c = f * c + i * gg
        h = o * jnp.tanh(c)
    deg = p0_ref[0, :, 0:1] + p1_ref[0, :, 0:1] + 1.0
    dinv = lax.rsqrt(deg)
    yw = jnp.dot(dinv * h, w1_ref[:], preferred_element_type=jnp.float32)
    yw_ref[0] = yw[:, 0:H // 2]
    yw_ref[1] = yw[:, H // 2:H]
    dinv_ref[:] = jnp.broadcast_to(dinv, (B, L))


def _lstm_call(x, wih_t, whh_t, bih, bhh, deg_parts, w1_t):
    return pl.pallas_call(
        _lstm_body,
        grid=(NBLK,),
        in_specs=[
            pl.BlockSpec((B, T, F), lambda i: (i, 0, 0)),
            pl.BlockSpec((F, 4 * H), lambda i: (0, 0)),
            pl.BlockSpec((H, 4 * H), lambda i: (0, 0)),
            pl.BlockSpec((1, 4 * H), lambda i: (0, 0)),
            pl.BlockSpec((1, 4 * H), lambda i: (0, 0)),
            pl.BlockSpec((1, B, L), lambda i: (0, i, 0)),
            pl.BlockSpec((1, B, L), lambda i: (1, i, 0)),
            pl.BlockSpec((H, H), lambda i: (0, 0)),
        ],
        out_specs=[
            pl.BlockSpec((2, B, H // 2), lambda i: (0, i, 0)),
            pl.BlockSpec((B, L), lambda i: (i, 0)),
        ],
        out_shape=[
            jax.ShapeDtypeStruct((2, N, H // 2), jnp.float32),
            jax.ShapeDtypeStruct((N, L), jnp.float32),
        ],
    )(x, wih_t, whh_t, bih, bhh, deg_parts, deg_parts, w1_t)


def _comb1_body(slo_ref, shi_ref, ylo_ref, yhi_ref, dinv_ref, b1_ref, w_ref, out_ref):
    dinv = dinv_ref[:, 0:1]
    zlo = jnp.maximum(dinv * (slo_ref[0] + ylo_ref[0]) + b1_ref[:, 0:H // 2], 0.0)
    zhi = jnp.maximum(dinv * (shi_ref[0] + yhi_ref[0]) + b1_ref[:, H // 2:H], 0.0)
    z = jnp.concatenate([zlo, zhi], axis=1)
    yw2 = jnp.dot(dinv * z, w_ref[:], preferred_element_type=jnp.float32)
    out_ref[0] = yw2[:, 0:H // 4]
    out_ref[1] = yw2[:, H // 4:H // 2]


def _comb1_call(seg1, yw1_s, dinv16, b1, w2_t):
    return pl.pallas_call(
        _comb1_body,
        grid=(NBLK,),
        in_specs=[
            pl.BlockSpec((1, B, H // 2), lambda i: (0, i, 0)),
            pl.BlockSpec((1, B, H // 2), lambda i: (1, i, 0)),
            pl.BlockSpec((1, B, H // 2), lambda i: (0, i, 0)),
            pl.BlockSpec((1, B, H // 2), lambda i: (1, i, 0)),
            pl.BlockSpec((B, L), lambda i: (i, 0)),
            pl.BlockSpec((1, H), lambda i: (0, 0)),
            pl.BlockSpec((H, H // 2), lambda i: (0, 0)),
        ],
        out_specs=pl.BlockSpec((2, B, H // 4), lambda i: (0, i, 0)),
        out_shape=jax.ShapeDtypeStruct((2, N, H // 4), jnp.float32),
    )(seg1, seg1, yw1_s, yw1_s, dinv16, b1, w2_t)


def _comb2_body(slo_ref, shi_ref, ylo_ref, yhi_ref, dinv_ref, b2_ref, wc_ref,
                bc_ref, out_ref):
    dinv = dinv_ref[:, 0:1]
    zlo = jnp.maximum(dinv * (slo_ref[0] + ylo_ref[0]) + b2_ref[:, 0:H // 4], 0.0)
    zhi = jnp.maximum(dinv * (shi_ref[0] + yhi_ref[0]) + b2_ref[:, H // 4:H // 2], 0.0)
    z = jnp.concatenate([zlo, zhi], axis=1)
    logits = jnp.dot(z, wc_ref[:], preferred_element_type=jnp.float32) + bc_ref[:]
    m = jnp.max(logits, axis=1, keepdims=True)
    e = jnp.exp(logits - m)
    out_ref[:] = e / jnp.sum(e, axis=1, keepdims=True)


def _comb2_call(seg2, yw2_s, dinv16, b2, wc_t, bc):
    return pl.pallas_call(
        _comb2_body,
        grid=(NBLK,),
        in_specs=[
            pl.BlockSpec((1, B, H // 4), lambda i: (0, i, 0)),
            pl.BlockSpec((1, B, H // 4), lambda i: (1, i, 0)),
            pl.BlockSpec((1, B, H // 4), lambda i: (0, i, 0)),
            pl.BlockSpec((1, B, H // 4), lambda i: (1, i, 0)),
            pl.BlockSpec((B, L), lambda i: (i, 0)),
            pl.BlockSpec((1, H // 2), lambda i: (0, 0)),
            pl.BlockSpec((H // 2, 2), lambda i: (0, 0)),
            pl.BlockSpec((1, 2), lambda i: (0, 0)),
        ],
        out_specs=pl.BlockSpec((B, 2), lambda i: (i, 0)),
        out_shape=jax.ShapeDtypeStruct((N, 2), jnp.float32),
    )(seg2, seg2, yw2_s, yw2_s, dinv16, b2, wc_t, bc)


def kernel(x, edge_index, W_ih, W_hh, b_ih, b_hh, W1, b1, W2, b2, Wc, bc):
    src = edge_index[0].astype(jnp.int32)
    dst = edge_index[1].astype(jnp.int32)
    pad = E_PAD - E
    pad_pos = jnp.arange(pad, dtype=jnp.int32)
    srcp = jnp.concatenate([src, pad_pos % N])
    dstp = jnp.concatenate([dst, N + pad_pos % (N_ACC - N)])
    dst3 = dstp.reshape(NS, NCH, CH)
    sr = srcp.reshape(1, NS, NCH, 1, CH)
    dr = jnp.broadcast_to(dstp.reshape(1, NS, NCH, 1, CH), (2, NS, NCH, 1, CH))
    off = jnp.array([0, N], jnp.int32).reshape(2, 1, 1, 1, 1)
    eidx = jnp.concatenate([sr + off, dr], axis=3)

    deg_parts = _deg_kernel(dst3)

    yw1_s, dinv16 = _lstm_call(x, W_ih.T, W_hh.T, b_ih.reshape(1, -1),
                               b_hh.reshape(1, -1), deg_parts, W1.T)
    seg1 = _seg_sum_32(yw1_s.reshape(2 * N, H // 2), eidx)
    yw2_s = _comb1_call(seg1, yw1_s, dinv16, b1.reshape(1, -1), W2.T)
    seg2 = _seg_sum_16(yw2_s.reshape(2 * N, H // 4), eidx)
    return _comb2_call(seg2, yw2_s, dinv16, b2.reshape(1, -1), Wc.T,
                       bc.reshape(1, -1))
